# baseline (speedup 1.0000x reference)
import functools

import jax
import jax.numpy as jnp
from jax import lax
from jax.experimental import pallas as pl
from jax.experimental.pallas import tpu as pltpu
from jax.experimental.pallas import tpu_sc as plsc

B, T, D, S, P, R = 8, 2048, 256, 64, 2048, 16
SPAN_WIN = 32
NC, NS, L = 2, 16, 16
NW = NC * NS
SC_B = 4
CHUNK = (SC_B * P) // NW
WPB = P // CHUNK


def _pool_spans(starts_ref, lens_ref, encoded_ref, bidx, pooled_ref):
    neg = jnp.finfo(jnp.float32).min
    row_id = lax.broadcasted_iota(jnp.int32, (SPAN_WIN + 8, D), 0)

    for k in range(S // 8):
        group = []
        for j in range(8):
            s = k * 8 + j
            start = starts_ref[bidx, s]
            ln = lens_ref[bidx, s]
            base = (start // 8) * 8
            off = start - base
            rows = encoded_ref[0, pl.ds(base, SPAN_WIN + 8), :]
            in_span = (row_id - off).astype(jnp.uint32) < ln.astype(jnp.uint32)
            masked = jnp.where(in_span, rows, neg)
            group.append(jnp.max(masked, axis=0, keepdims=True))
        pooled_ref[k * 8:(k + 1) * 8, :] = jnp.concatenate(group, axis=0)


def _pool_project_kernel(starts_ref, lens_ref, encoded_ref, w_ref, b_ref,
                         h_ref, t_ref, pooled_ref, *, boff):
    bidx = pl.program_id(0) + boff
    _pool_spans(starts_ref, lens_ref, encoded_ref, bidx, pooled_ref)
    pooled = pooled_ref[...]
    h = jnp.dot(pooled, w_ref[:D, :], preferred_element_type=jnp.float32)
    h_ref[0, :, :] = h + b_ref[...][None, :]
    t_ref[0, :, :] = jnp.dot(pooled, w_ref[D:, :],
                             preferred_element_type=jnp.float32)


def _pool_project_score_kernel(starts_ref, lens_ref, encoded_ref, w_ref, b_ref,
                               head_ref, tail_ref, sc_ref, out_ref, pooled_ref):
    del sc_ref
    bidx = pl.program_id(0)
    _pool_spans(starts_ref, lens_ref, encoded_ref, bidx, pooled_ref)
    pooled = pooled_ref[...]
    h = jnp.dot(pooled, w_ref[:D, :],
                preferred_element_type=jnp.float32) + b_ref[...][None, :]
    t = jnp.dot(pooled, w_ref[D:, :], preferred_element_type=jnp.float32)
    ht = jnp.concatenate([h, t], axis=0)
    span_id = lax.broadcasted_iota(jnp.int32, (P, 2 * S), 1)
    sel = ((span_id == head_ref[bidx, :][:, None])
           | (span_id == tail_ref[bidx, :][:, None] + S))
    out_ref[0] = jnp.dot(sel.astype(jnp.float32), ht,
                         preferred_element_type=jnp.float32)


def _pair_gather_kernel(h_hbm, t_hbm, head_hbm, tail_hbm, out_hbm,
                        h_v, t_v, idxh_v, idxt_v, out_v,
                        sem0, sem1, sem2, sem3):
    wid = lax.axis_index("s") * NC + lax.axis_index("c")
    bidx = wid // WPB
    src_b = bidx + (B - SC_B)
    col = pl.multiple_of((wid % WPB) * CHUNK, CHUNK)
    cps = (pltpu.async_copy(h_hbm.at[bidx], h_v, sem0),
           pltpu.async_copy(t_hbm.at[bidx], t_v, sem1),
           pltpu.async_copy(head_hbm.at[src_b, pl.ds(col, CHUNK)], idxh_v, sem2),
           pltpu.async_copy(tail_hbm.at[src_b, pl.ds(col, CHUNK)], idxt_v, sem3))
    for cp in cps:
        cp.wait()

    def group_body(g, _):
        off = pl.multiple_of(g * L, L)
        rows_h = idxh_v[pl.ds(off, L)]
        rows_t = idxt_v[pl.ds(off, L)]
        for j in range(L):
            i = off + j
            vh = h_v[rows_h[j], :]
            vt = t_v[rows_t[j], :]
            out_v[i, :] = vh + vt
        return 0

    lax.fori_loop(0, CHUNK // L, group_body, 0)

    pltpu.sync_copy(out_v, out_hbm.at[src_b, pl.ds(col, CHUNK), :])


def _onehot_score_kernel(h_ref, t_ref, head_ref, tail_ref, out_ref):
    bidx = pl.program_id(0)
    span_id = lax.broadcasted_iota(jnp.int32, (P, S), 1)
    oh_h = (head_ref[bidx, :][:, None] == span_id).astype(jnp.float32)
    oh_t = (tail_ref[bidx, :][:, None] == span_id).astype(jnp.float32)
    out_ref[0] = (jnp.dot(oh_h, h_ref[0], preferred_element_type=jnp.float32)
                  + jnp.dot(oh_t, t_ref[0], preferred_element_type=jnp.float32))


def kernel(encoded, span_starts, span_lengths, pair_head, pair_tail, W, b):
    TC_B = B - SC_B
    starts32 = span_starts.astype(jnp.int32)
    lens32 = span_lengths.astype(jnp.int32)
    head32 = pair_head.astype(jnp.int32)
    tail32 = pair_tail.astype(jnp.int32)

    sc_grid_spec = pltpu.PrefetchScalarGridSpec(
        num_scalar_prefetch=2,
        grid=(SC_B,),
        in_specs=[
            pl.BlockSpec((1, T, D), lambda b_, *_: (b_ + B - SC_B, 0, 0)),
            pl.BlockSpec((2 * D, R), lambda b_, *_: (0, 0)),
            pl.BlockSpec((R,), lambda b_, *_: (0,)),
        ],
        out_specs=[
            pl.BlockSpec((1, S, R), lambda b_, *_: (b_, 0, 0)),
            pl.BlockSpec((1, S, R), lambda b_, *_: (b_, 0, 0)),
        ],
        scratch_shapes=[pltpu.VMEM((S, D), jnp.float32)],
    )
    h, t = pl.pallas_call(
        functools.partial(_pool_project_kernel, boff=TC_B),
        grid_spec=sc_grid_spec,
        out_shape=[
            jax.ShapeDtypeStruct((SC_B, S, R), jnp.float32),
            jax.ShapeDtypeStruct((SC_B, S, R), jnp.float32),
        ],
    )(starts32, lens32, encoded, W, b)

    mesh = plsc.VectorSubcoreMesh(core_axis_name="c", subcore_axis_name="s",
                                  num_cores=NC)
    gather = pl.kernel(
        _pair_gather_kernel,
        out_type=jax.ShapeDtypeStruct((B, P, R), jnp.float32),
        mesh=mesh,
        compiler_params=pltpu.CompilerParams(needs_layout_passes=False),
        scratch_types=[
            pltpu.VMEM((S, R), jnp.float32),
            pltpu.VMEM((S, R), jnp.float32),
            pltpu.VMEM((CHUNK,), jnp.int32),
            pltpu.VMEM((CHUNK,), jnp.int32),
            pltpu.VMEM((CHUNK, R), jnp.float32),
            pltpu.SemaphoreType.DMA,
            pltpu.SemaphoreType.DMA,
            pltpu.SemaphoreType.DMA,
            pltpu.SemaphoreType.DMA,
        ],
    )
    sc_scores = gather(h, t, head32, tail32)

    tc_grid_spec = pltpu.PrefetchScalarGridSpec(
        num_scalar_prefetch=2,
        grid=(TC_B,),
        in_specs=[
            pl.BlockSpec((1, T, D), lambda b_, *_: (b_, 0, 0)),
            pl.BlockSpec((2 * D, R), lambda b_, *_: (0, 0)),
            pl.BlockSpec((R,), lambda b_, *_: (0,)),
            pl.BlockSpec((TC_B, P), lambda b_, *_: (0, 0)),
            pl.BlockSpec((TC_B, P), lambda b_, *_: (0, 0)),
            pl.BlockSpec(memory_space=pl.ANY),
        ],
        out_specs=pl.BlockSpec((1, P, R), lambda b_, *_: (b_, 0, 0)),
        scratch_shapes=[pltpu.VMEM((S, D), jnp.float32)],
    )
    return pl.pallas_call(
        _pool_project_score_kernel,
        grid_spec=tc_grid_spec,
        out_shape=jax.ShapeDtypeStruct((B, P, R), jnp.float32),
        input_output_aliases={7: 0},
    )(starts32, lens32, encoded, W, b, head32[:TC_B], tail32[:TC_B],
      sc_scores)


# speedup vs baseline: 1.0832x; 1.0832x over previous
import functools

import jax
import jax.numpy as jnp
from jax import lax
from jax.experimental import pallas as pl
from jax.experimental.pallas import tpu as pltpu
from jax.experimental.pallas import tpu_sc as plsc

B, T, D, S, P, R = 8, 2048, 256, 64, 2048, 16
SPAN_WIN = 32
NC, NS, L = 2, 16, 16
NW = NC * NS
SC_B = 4
CHUNK = (SC_B * P) // NW
WPB = P // CHUNK


def _pool_spans(starts_ref, lens_ref, encoded_ref, bidx, pooled_ref):
    neg = jnp.finfo(jnp.float32).min
    row_id = lax.broadcasted_iota(jnp.int32, (SPAN_WIN + 8, D), 0)

    for k in range(S // 8):
        group = []
        for j in range(8):
            s = k * 8 + j
            start = starts_ref[bidx, s]
            ln = lens_ref[bidx, s]
            base = (start // 8) * 8
            off = start - base
            rows = encoded_ref[0, pl.ds(base, SPAN_WIN + 8), :]
            in_span = (row_id - off).astype(jnp.uint32) < ln.astype(jnp.uint32)
            masked = jnp.where(in_span, rows, neg)
            group.append(jnp.max(masked, axis=0, keepdims=True))
        pooled_ref[k * 8:(k + 1) * 8, :] = jnp.concatenate(group, axis=0)


def _pool_project_kernel(starts_ref, lens_ref, encoded_ref, w_ref, b_ref,
                         h_ref, t_ref, pooled_ref, *, boff):
    bidx = pl.program_id(0) + boff
    _pool_spans(starts_ref, lens_ref, encoded_ref, bidx, pooled_ref)
    pooled = pooled_ref[...]
    h = jnp.dot(pooled, w_ref[:D, :], preferred_element_type=jnp.float32)
    h_ref[0, :, :] = h + b_ref[...][None, :]
    t_ref[0, :, :] = jnp.dot(pooled, w_ref[D:, :],
                             preferred_element_type=jnp.float32)


def _pool_project_score_kernel(starts_ref, lens_ref, encoded_ref, w_ref, b_ref,
                               head_ref, tail_ref, out_ref, pooled_ref):
    bidx = pl.program_id(0)
    _pool_spans(starts_ref, lens_ref, encoded_ref, bidx, pooled_ref)
    pooled = pooled_ref[...]
    h = jnp.dot(pooled, w_ref[:D, :],
                preferred_element_type=jnp.float32) + b_ref[...][None, :]
    t = jnp.dot(pooled, w_ref[D:, :], preferred_element_type=jnp.float32)
    ht = jnp.concatenate([h, t], axis=0)
    span_id = lax.broadcasted_iota(jnp.int32, (P, 2 * S), 1)
    sel = ((span_id == head_ref[bidx, :][:, None])
           | (span_id == tail_ref[bidx, :][:, None] + S))
    out_ref[0] = jnp.dot(sel.astype(jnp.float32), ht,
                         preferred_element_type=jnp.float32)


def _pair_gather_kernel(h_hbm, t_hbm, head_hbm, tail_hbm, out_hbm,
                        h_v, t_v, idxh_v, idxt_v, out_v,
                        sem0, sem1, sem2, sem3):
    wid = lax.axis_index("s") * NC + lax.axis_index("c")
    bidx = wid // WPB
    src_b = bidx + (B - SC_B)
    col = pl.multiple_of((wid % WPB) * CHUNK, CHUNK)
    cps = (pltpu.async_copy(h_hbm.at[bidx], h_v, sem0),
           pltpu.async_copy(t_hbm.at[bidx], t_v, sem1),
           pltpu.async_copy(head_hbm.at[src_b, pl.ds(col, CHUNK)], idxh_v, sem2),
           pltpu.async_copy(tail_hbm.at[src_b, pl.ds(col, CHUNK)], idxt_v, sem3))
    for cp in cps:
        cp.wait()

    def group_body(g, _):
        off = pl.multiple_of(g * L, L)
        rows_h = idxh_v[pl.ds(off, L)]
        rows_t = idxt_v[pl.ds(off, L)]
        for j in range(L):
            i = off + j
            vh = h_v[rows_h[j], :]
            vt = t_v[rows_t[j], :]
            out_v[i, :] = vh + vt
        return 0

    lax.fori_loop(0, CHUNK // L, group_body, 0)

    pltpu.sync_copy(out_v, out_hbm.at[bidx, pl.ds(col, CHUNK), :])


def _onehot_score_kernel(h_ref, t_ref, head_ref, tail_ref, out_ref):
    bidx = pl.program_id(0)
    span_id = lax.broadcasted_iota(jnp.int32, (P, S), 1)
    oh_h = (head_ref[bidx, :][:, None] == span_id).astype(jnp.float32)
    oh_t = (tail_ref[bidx, :][:, None] == span_id).astype(jnp.float32)
    out_ref[0] = (jnp.dot(oh_h, h_ref[0], preferred_element_type=jnp.float32)
                  + jnp.dot(oh_t, t_ref[0], preferred_element_type=jnp.float32))


def kernel(encoded, span_starts, span_lengths, pair_head, pair_tail, W, b):
    TC_B = B - SC_B
    starts32 = span_starts.astype(jnp.int32)
    lens32 = span_lengths.astype(jnp.int32)
    head32 = pair_head.astype(jnp.int32)
    tail32 = pair_tail.astype(jnp.int32)

    sc_grid_spec = pltpu.PrefetchScalarGridSpec(
        num_scalar_prefetch=2,
        grid=(SC_B,),
        in_specs=[
            pl.BlockSpec((1, T, D), lambda b_, *_: (b_ + B - SC_B, 0, 0)),
            pl.BlockSpec((2 * D, R), lambda b_, *_: (0, 0)),
            pl.BlockSpec((R,), lambda b_, *_: (0,)),
        ],
        out_specs=[
            pl.BlockSpec((1, S, R), lambda b_, *_: (b_, 0, 0)),
            pl.BlockSpec((1, S, R), lambda b_, *_: (b_, 0, 0)),
        ],
        scratch_shapes=[pltpu.VMEM((S, D), jnp.float32)],
    )
    h, t = pl.pallas_call(
        functools.partial(_pool_project_kernel, boff=TC_B),
        grid_spec=sc_grid_spec,
        out_shape=[
            jax.ShapeDtypeStruct((SC_B, S, R), jnp.float32),
            jax.ShapeDtypeStruct((SC_B, S, R), jnp.float32),
        ],
    )(starts32, lens32, encoded, W, b)

    mesh = plsc.VectorSubcoreMesh(core_axis_name="c", subcore_axis_name="s",
                                  num_cores=NC)
    gather = pl.kernel(
        _pair_gather_kernel,
        out_type=jax.ShapeDtypeStruct((SC_B, P, R), jnp.float32),
        mesh=mesh,
        compiler_params=pltpu.CompilerParams(needs_layout_passes=False),
        scratch_types=[
            pltpu.VMEM((S, R), jnp.float32),
            pltpu.VMEM((S, R), jnp.float32),
            pltpu.VMEM((CHUNK,), jnp.int32),
            pltpu.VMEM((CHUNK,), jnp.int32),
            pltpu.VMEM((CHUNK, R), jnp.float32),
            pltpu.SemaphoreType.DMA,
            pltpu.SemaphoreType.DMA,
            pltpu.SemaphoreType.DMA,
            pltpu.SemaphoreType.DMA,
        ],
    )
    sc_scores = gather(h, t, head32, tail32)

    tc_grid_spec = pltpu.PrefetchScalarGridSpec(
        num_scalar_prefetch=2,
        grid=(TC_B,),
        in_specs=[
            pl.BlockSpec((1, T, D), lambda b_, *_: (b_, 0, 0)),
            pl.BlockSpec((2 * D, R), lambda b_, *_: (0, 0)),
            pl.BlockSpec((R,), lambda b_, *_: (0,)),
            pl.BlockSpec((TC_B, P), lambda b_, *_: (0, 0)),
            pl.BlockSpec((TC_B, P), lambda b_, *_: (0, 0)),
        ],
        out_specs=pl.BlockSpec((1, P, R), lambda b_, *_: (b_, 0, 0)),
        scratch_shapes=[pltpu.VMEM((S, D), jnp.float32)],
    )
    tc_scores = pl.pallas_call(
        _pool_project_score_kernel,
        grid_spec=tc_grid_spec,
        out_shape=jax.ShapeDtypeStruct((TC_B, P, R), jnp.float32),
    )(starts32, lens32, encoded, W, b, head32[:TC_B], tail32[:TC_B])
    return jnp.concatenate([tc_scores, sc_scores], axis=0)
